# baseline (speedup 1.0000x reference)
import numpy as np
import jax
import jax.numpy as jnp
from jax import lax
from jax.experimental import pallas as pl
from jax.experimental.pallas import tpu as pltpu
from jax.experimental.pallas import tpu_sc as plsc

C = 128
K = 15
CPG = 16
GROUPS = 8
H = 32
SIGMA = 2.0
N = 10000
M = 10000

NC = 2
NS = 16
NW = NC * NS
QB = 8
NCHUNK = M // QB
MODW = 256


def _interleave_pairs(x):
    b = x.astype(jnp.bfloat16).reshape(x.shape[:-1] + (4, 2, 16))
    b = jnp.swapaxes(b, -1, -2)
    w = lax.bitcast_convert_type(b, jnp.int32)
    return w.reshape(x.shape[:-1] + (64,))


def _mlp_body(x_ref, w1_ref, b1_ref, w2_ref, o_ref, o2_ref):
    x = x_ref[...]
    mb = x.shape[0]
    h = jnp.dot(x, w1_ref[...], preferred_element_type=jnp.float32)
    h = h + b1_ref[...]
    h = jnp.where(h > 0, h, h * 0.1)
    z = jnp.dot(h, w2_ref[...], preferred_element_type=jnp.float32)
    o_ref[...] = 1.0 / (1.0 + jnp.exp(-z))
    u = lax.bitcast_convert_type(x.astype(jnp.bfloat16), jnp.uint16)
    r = u.reshape(mb, 4, 2, 16)
    lo = r[:, :, 0, :].astype(jnp.uint32)
    hi = r[:, :, 1, :].astype(jnp.uint32)
    pair = (lo | (hi << 16)).reshape(mb, 64)
    o2_ref[...] = jnp.concatenate(
        [pair, jnp.zeros((mb, 64), jnp.uint32)], axis=1).astype(jnp.int32)


def _modulations(s_feats, W1, b1, W2p):
    mb = 1000
    return pl.pallas_call(
        _mlp_body,
        grid=(M // mb,),
        in_specs=[
            pl.BlockSpec((mb, C), lambda i: (i, 0)),
            pl.BlockSpec((C, C), lambda i: (0, 0)),
            pl.BlockSpec((1, C), lambda i: (0, 0)),
            pl.BlockSpec((C, MODW), lambda i: (0, 0)),
        ],
        out_specs=[
            pl.BlockSpec((mb, MODW), lambda i: (i, 0)),
            pl.BlockSpec((mb, C), lambda i: (i, 0)),
        ],
        out_shape=[
            jax.ShapeDtypeStruct((M, MODW), jnp.float32),
            jax.ShapeDtypeStruct((M, C), jnp.int32),
        ],
    )(s_feats, W1, b1.reshape(1, C), W2p)


def _take_splat(vec, idx):
    return vec.at[jnp.full((16,), 0, jnp.int32) + idx].get(
        mode="promise_in_bounds")


def _sc_body(nb_hbm, qp_hbm, spts_hbm, sfeats_hbm, wts_hbm, kp_hbm, mod_hbm,
             out_hbm,
             spts_v, wts_v, kp_v, nb_v, qp_v, mod_v, feat_v, out_v,
             sem_i, sem_f, sem_o):
    wid = lax.axis_index("c") * NS + lax.axis_index("s")

    pltpu.sync_copy(spts_hbm, spts_v)
    pltpu.sync_copy(wts_hbm, wts_v)
    pltpu.sync_copy(kp_hbm, kp_v)

    kpxv = kp_v[pl.ds(0, 16)]
    kpyv = kp_v[pl.ds(16, 16)]
    kpzv = kp_v[pl.ds(32, 16)]
    kpx = [kpxv[k] for k in range(K)]
    kpy = [kpyv[k] for k in range(K)]
    kpz = [kpzv[k] for k in range(K)]

    nchunks = (NCHUNK - wid + NW - 1) // NW

    def issue_idx(c):
        s = c % 3
        base = (wid + NW * c) * QB
        pltpu.async_copy(nb_hbm.at[pl.ds(base * H, QB * H)], nb_v.at[s],
                         sem_i)
        pltpu.async_copy(qp_hbm.at[pl.ds(base * 4, QB * 4)], qp_v.at[s],
                         sem_i)
        pltpu.async_copy(mod_hbm.at[pl.ds(base * MODW, QB * MODW)],
                         mod_v.at[s], sem_i)

    def wait_idx(c):
        s = c % 3
        pltpu.make_async_copy(nb_hbm.at[pl.ds(0, QB * H)], nb_v.at[s],
                              sem_i).wait()
        pltpu.make_async_copy(qp_hbm.at[pl.ds(0, QB * 4)], qp_v.at[s],
                              sem_i).wait()
        pltpu.make_async_copy(mod_hbm.at[pl.ds(0, QB * MODW)], mod_v.at[s],
                              sem_i).wait()

    def issue_feat(c):
        s, sf = c % 3, c % 2
        pltpu.async_copy(sfeats_hbm.at[nb_v.at[s, pl.ds(0, 128)]],
                         feat_v.at[sf, pl.ds(0, 128)], sem_f)
        pltpu.async_copy(sfeats_hbm.at[nb_v.at[s, pl.ds(128, 128)]],
                         feat_v.at[sf, pl.ds(128, 128)], sem_f)

    def wait_feat(c):
        s, sf = c % 3, c % 2
        pltpu.make_async_copy(sfeats_hbm.at[nb_v.at[s, pl.ds(0, 128)]],
                              feat_v.at[sf, pl.ds(0, 128)], sem_f).wait()
        pltpu.make_async_copy(sfeats_hbm.at[nb_v.at[s, pl.ds(128, 128)]],
                              feat_v.at[sf, pl.ds(128, 128)], sem_f).wait()

    def wait_out(c):
        base = (wid + NW * c) * QB
        pltpu.make_async_copy(out_v, out_hbm.at[pl.ds(base, QB)],
                              sem_o).wait()

    issue_idx(0)

    @pl.when(nchunks > 1)
    def _():
        issue_idx(1)

    wait_idx(0)
    issue_feat(0)

    def chunk_body(c, carry):
        sf = c % 2
        si = c % 3
        base = (wid + NW * c) * QB
        wait_feat(c)

        @pl.when(c + 1 < nchunks)
        def _():
            wait_idx(c + 1)

        @pl.when(c + 2 < nchunks)
        def _():
            issue_idx(c + 2)

        @pl.when(c + 1 < nchunks)
        def _():
            issue_feat(c + 1)

        @pl.when(c > 0)
        def _():
            wait_out(c - 1)

        qall0 = qp_v[si, pl.ds(0, 16)]
        qall1 = qp_v[si, pl.ds(16, 16)]

        def q_body(i, carry_q):
            qv = jnp.where(i < 4, qall0, qall1)
            qo = (i % 4) * 4
            qxs = _take_splat(qv, qo)
            qys = _take_splat(qv, qo + 1)
            qzs = _take_splat(qv, qo + 2)

            def half_body(jh, accs):
                t16 = i * H + jh * 16
                idx16 = nb_v[si, pl.ds(t16, 16)]
                xs = plsc.load_gather(spts_v, [idx16])
                ys = plsc.load_gather(spts_v, [idx16 + N])
                zs = plsc.load_gather(spts_v, [idx16 + 2 * N])
                dx = xs - qxs
                dy = ys - qys
                dz = zs - qzs
                best = jnp.full((16,), 1e30, jnp.float32)
                bestk = jnp.zeros((16,), jnp.int32)
                for k in range(K):
                    ddx = dx - kpx[k]
                    ddy = dy - kpy[k]
                    ddz = dz - kpz[k]
                    d2 = ddx * ddx + ddy * ddy + ddz * ddz
                    m = d2 < best
                    best = jnp.where(m, d2, best)
                    bestk = jnp.where(m, jnp.int32(k), bestk)
                x = jnp.maximum(best, jnp.float32(1e-24))
                xi = plsc.bitcast(x, jnp.int32)
                r = plsc.bitcast(jnp.int32(0x5F3759DF) - (xi >> 1),
                                 jnp.float32)
                for _ in range(3):
                    r = r * (1.5 - 0.5 * x * r * r)
                s = x * r
                infl16 = jnp.maximum(1.0 - s * jnp.float32(1.0 / SIGMA), 0.0)

                accs = list(accs)
                for l in range(16):
                    kk = bestk[l]
                    fl = infl16[l]
                    m16 = mod_v[si, pl.ds(i * MODW + kk * CPG, 16)]
                    modi = m16 * fl
                    n = t16 + l
                    for b in range(4):
                        fv = plsc.bitcast(feat_v[sf, n, pl.ds(b * 16, 16)],
                                          jnp.bfloat16)
                        wv = plsc.bitcast(
                            wts_v[pl.ds(kk * 64 + b * 16, 16)],
                            jnp.bfloat16)
                        te, to = plsc.unpack(
                            fv * wv, format=plsc.PackFormat.INTERLEAVED)
                        accs[2 * b] = accs[2 * b] + te * modi
                        accs[2 * b + 1] = accs[2 * b + 1] + to * modi
                return tuple(accs)

            accs = lax.fori_loop(
                0, 2, half_body,
                tuple(jnp.zeros((16,), jnp.float32) for _ in range(GROUPS)))
            for g in range(GROUPS):
                out_v[i, pl.ds(g * CPG, 16)] = accs[g]
            return carry_q

        lax.fori_loop(0, QB, q_body, 0)
        pltpu.async_copy(out_v, out_hbm.at[pl.ds(base, QB)], sem_o)
        return carry

    lax.fori_loop(0, nchunks, chunk_body, 0)
    wait_out(nchunks - 1)


def _sc_main(nb_flat, qp_pad, spts_flat, s_feats, wts_flat, kp, mod_flat):
    mesh = plsc.VectorSubcoreMesh(core_axis_name="c", subcore_axis_name="s",
                                  num_cores=NC, num_subcores=NS)
    return pl.kernel(
        _sc_body,
        out_type=jax.ShapeDtypeStruct((M, C), jnp.float32),
        mesh=mesh,
        compiler_params=pltpu.CompilerParams(needs_layout_passes=False),
        scratch_types=[
            pltpu.VMEM((3 * N,), jnp.float32),
            pltpu.VMEM((K * 64,), jnp.int32),
            pltpu.VMEM((48,), jnp.float32),
            pltpu.VMEM((3, QB * H), jnp.int32),
            pltpu.VMEM((3, QB * 4), jnp.float32),
            pltpu.VMEM((3, QB * MODW), jnp.float32),
            pltpu.VMEM((2, QB * H, C), jnp.int32),
            pltpu.VMEM((QB, C), jnp.float32),
            pltpu.SemaphoreType.DMA,
            pltpu.SemaphoreType.DMA,
            pltpu.SemaphoreType.DMA,
        ],
    )(nb_flat, qp_pad, spts_flat, s_feats, wts_flat, kp, mod_flat)


def kernel(q_pts, s_pts, s_feats, neighb_inds, weights, W1, b1, W2,
           kernel_points):
    nb_flat = neighb_inds.astype(jnp.int32).reshape(-1)
    qp_flat = jnp.pad(q_pts, ((0, 0), (0, 1))).reshape(-1)
    spts_flat = s_pts.T.reshape(-1)
    wts_flat = _interleave_pairs(weights).reshape(-1)
    kp_flat = jnp.pad(kernel_points, ((0, 16 - K), (0, 0))).T.reshape(-1)
    W2p = jnp.pad(W2, ((0, 0), (0, MODW - K * CPG)))
    mod, sf_pairs = _modulations(s_feats, W1, b1, W2p)
    return _sc_main(nb_flat, qp_flat, spts_flat, sf_pairs, wts_flat,
                    kp_flat, mod.reshape(-1))


# speedup vs baseline: 1.2124x; 1.2124x over previous
import numpy as np
import jax
import jax.numpy as jnp
from jax import lax
from jax.experimental import pallas as pl
from jax.experimental.pallas import tpu as pltpu
from jax.experimental.pallas import tpu_sc as plsc

C = 128
K = 15
CPG = 16
GROUPS = 8
H = 32
SIGMA = 2.0
N = 10000
M = 10000

NC = 2
NS = 16
NW = NC * NS
QB = 8
NCHUNK = M // QB
MODW = 256


def _interleave_pairs(x):
    b = x.astype(jnp.bfloat16).reshape(x.shape[:-1] + (4, 2, 16))
    b = jnp.swapaxes(b, -1, -2)
    w = lax.bitcast_convert_type(b, jnp.int32)
    return w.reshape(x.shape[:-1] + (64,))


def _mlp_body(x_ref, w1_ref, b1_ref, w2_ref, o_ref):
    x = x_ref[...]
    h = jnp.dot(x, w1_ref[...], preferred_element_type=jnp.float32)
    h = h + b1_ref[...]
    h = jnp.where(h > 0, h, h * 0.1)
    z = jnp.dot(h, w2_ref[...], preferred_element_type=jnp.float32)
    o_ref[...] = 1.0 / (1.0 + jnp.exp(-z))


def _modulations(s_feats, W1, b1, W2p):
    mb = 1000
    return pl.pallas_call(
        _mlp_body,
        grid=(M // mb,),
        in_specs=[
            pl.BlockSpec((mb, C), lambda i: (i, 0)),
            pl.BlockSpec((C, C), lambda i: (0, 0)),
            pl.BlockSpec((1, C), lambda i: (0, 0)),
            pl.BlockSpec((C, MODW), lambda i: (0, 0)),
        ],
        out_specs=pl.BlockSpec((mb, MODW), lambda i: (i, 0)),
        out_shape=jax.ShapeDtypeStruct((M, MODW), jnp.float32),
    )(s_feats, W1, b1.reshape(1, C), W2p)


def _take_splat(vec, idx):
    return vec.at[jnp.full((16,), 0, jnp.int32) + idx].get(
        mode="promise_in_bounds")


def _sc_body(nb_hbm, qp_hbm, spts_hbm, sfeats_hbm, wts_hbm, kp_hbm, mod_hbm,
             out_hbm,
             spts_v, wts_v, kp_v, nb_v, qp_v, mod_v, feat_v, out_v,
             sem_i, sem_f, sem_o):
    wid = lax.axis_index("c") * NS + lax.axis_index("s")

    pltpu.sync_copy(spts_hbm, spts_v)
    pltpu.sync_copy(wts_hbm, wts_v)
    pltpu.sync_copy(kp_hbm, kp_v)

    kpxv = kp_v[pl.ds(0, 16)]
    kpyv = kp_v[pl.ds(16, 16)]
    kpzv = kp_v[pl.ds(32, 16)]
    kpx = [kpxv[k] for k in range(K)]
    kpy = [kpyv[k] for k in range(K)]
    kpz = [kpzv[k] for k in range(K)]

    nchunks = (NCHUNK - wid + NW - 1) // NW

    def issue_idx(c):
        s = c % 3
        base = (wid + NW * c) * QB
        pltpu.async_copy(nb_hbm.at[pl.ds(base * H, QB * H)], nb_v.at[s],
                         sem_i)
        pltpu.async_copy(qp_hbm.at[pl.ds(base * 4, QB * 4)], qp_v.at[s],
                         sem_i)
        pltpu.async_copy(mod_hbm.at[pl.ds(base * MODW, QB * MODW)],
                         mod_v.at[s], sem_i)

    def wait_idx(c):
        s = c % 3
        pltpu.make_async_copy(nb_hbm.at[pl.ds(0, QB * H)], nb_v.at[s],
                              sem_i).wait()
        pltpu.make_async_copy(qp_hbm.at[pl.ds(0, QB * 4)], qp_v.at[s],
                              sem_i).wait()
        pltpu.make_async_copy(mod_hbm.at[pl.ds(0, QB * MODW)], mod_v.at[s],
                              sem_i).wait()

    def issue_feat(c):
        s, sf = c % 3, c % 2
        pltpu.async_copy(sfeats_hbm.at[nb_v.at[s, pl.ds(0, 128)]],
                         feat_v.at[sf, pl.ds(0, 128)], sem_f)
        pltpu.async_copy(sfeats_hbm.at[nb_v.at[s, pl.ds(128, 128)]],
                         feat_v.at[sf, pl.ds(128, 128)], sem_f)

    def wait_feat(c):
        s, sf = c % 3, c % 2
        pltpu.make_async_copy(sfeats_hbm.at[nb_v.at[s, pl.ds(0, 128)]],
                              feat_v.at[sf, pl.ds(0, 128)], sem_f).wait()
        pltpu.make_async_copy(sfeats_hbm.at[nb_v.at[s, pl.ds(128, 128)]],
                              feat_v.at[sf, pl.ds(128, 128)], sem_f).wait()

    def wait_out(c):
        base = (wid + NW * c) * QB
        pltpu.make_async_copy(out_v, out_hbm.at[pl.ds(base, QB)],
                              sem_o).wait()

    issue_idx(0)

    @pl.when(nchunks > 1)
    def _():
        issue_idx(1)

    wait_idx(0)
    issue_feat(0)

    def chunk_body(c, carry):
        sf = c % 2
        si = c % 3
        base = (wid + NW * c) * QB
        wait_feat(c)

        @pl.when(c + 1 < nchunks)
        def _():
            wait_idx(c + 1)

        @pl.when(c + 2 < nchunks)
        def _():
            issue_idx(c + 2)

        @pl.when(c + 1 < nchunks)
        def _():
            issue_feat(c + 1)

        @pl.when(c > 0)
        def _():
            wait_out(c - 1)

        qall0 = qp_v[si, pl.ds(0, 16)]
        qall1 = qp_v[si, pl.ds(16, 16)]

        def q_body(i, carry_q):
            qv = jnp.where(i < 4, qall0, qall1)
            qo = (i % 4) * 4
            qxs = _take_splat(qv, qo)
            qys = _take_splat(qv, qo + 1)
            qzs = _take_splat(qv, qo + 2)

            def half_body(jh, accs):
                t16 = i * H + jh * 16
                idx16 = nb_v[si, pl.ds(t16, 16)]
                xs = plsc.load_gather(spts_v, [idx16])
                ys = plsc.load_gather(spts_v, [idx16 + N])
                zs = plsc.load_gather(spts_v, [idx16 + 2 * N])
                dx = xs - qxs
                dy = ys - qys
                dz = zs - qzs
                best = jnp.full((16,), 1e30, jnp.float32)
                bestk = jnp.zeros((16,), jnp.int32)
                for k in range(K):
                    ddx = dx - kpx[k]
                    ddy = dy - kpy[k]
                    ddz = dz - kpz[k]
                    d2 = ddx * ddx + ddy * ddy + ddz * ddz
                    m = d2 < best
                    best = jnp.where(m, d2, best)
                    bestk = jnp.where(m, jnp.int32(k), bestk)
                x = jnp.maximum(best, jnp.float32(1e-24))
                xi = plsc.bitcast(x, jnp.int32)
                r = plsc.bitcast(jnp.int32(0x5F3759DF) - (xi >> 1),
                                 jnp.float32)
                for _ in range(3):
                    r = r * (1.5 - 0.5 * x * r * r)
                s = x * r
                infl16 = jnp.maximum(1.0 - s * jnp.float32(1.0 / SIGMA), 0.0)

                accs = list(accs)
                for l in range(16):
                    kk = bestk[l]
                    fl = infl16[l]
                    m16 = mod_v[si, pl.ds(i * MODW + kk * CPG, 16)]
                    modi = m16 * fl
                    n = t16 + l
                    for b in range(4):
                        fv = plsc.bitcast(feat_v[sf, n, pl.ds(b * 16, 16)],
                                          jnp.bfloat16)
                        wv = plsc.bitcast(
                            wts_v[pl.ds(kk * 64 + b * 16, 16)],
                            jnp.bfloat16)
                        te, to = plsc.unpack(
                            fv * wv, format=plsc.PackFormat.INTERLEAVED)
                        accs[2 * b] = accs[2 * b] + te * modi
                        accs[2 * b + 1] = accs[2 * b + 1] + to * modi
                return tuple(accs)

            accs = lax.fori_loop(
                0, 2, half_body,
                tuple(jnp.zeros((16,), jnp.float32) for _ in range(GROUPS)))
            for g in range(GROUPS):
                out_v[i, pl.ds(g * CPG, 16)] = accs[g]
            return carry_q

        lax.fori_loop(0, QB, q_body, 0)
        pltpu.async_copy(out_v, out_hbm.at[pl.ds(base, QB)], sem_o)
        return carry

    lax.fori_loop(0, nchunks, chunk_body, 0)
    wait_out(nchunks - 1)


def _sc_main(nb_flat, qp_pad, spts_flat, s_feats, wts_flat, kp, mod_flat):
    mesh = plsc.VectorSubcoreMesh(core_axis_name="c", subcore_axis_name="s",
                                  num_cores=NC, num_subcores=NS)
    return pl.kernel(
        _sc_body,
        out_type=jax.ShapeDtypeStruct((M, C), jnp.float32),
        mesh=mesh,
        compiler_params=pltpu.CompilerParams(needs_layout_passes=False),
        scratch_types=[
            pltpu.VMEM((3 * N,), jnp.float32),
            pltpu.VMEM((K * 64,), jnp.int32),
            pltpu.VMEM((48,), jnp.float32),
            pltpu.VMEM((3, QB * H), jnp.int32),
            pltpu.VMEM((3, QB * 4), jnp.float32),
            pltpu.VMEM((3, QB * MODW), jnp.float32),
            pltpu.VMEM((2, QB * H, C), jnp.int32),
            pltpu.VMEM((QB, C), jnp.float32),
            pltpu.SemaphoreType.DMA,
            pltpu.SemaphoreType.DMA,
            pltpu.SemaphoreType.DMA,
        ],
    )(nb_flat, qp_pad, spts_flat, s_feats, wts_flat, kp, mod_flat)


def kernel(q_pts, s_pts, s_feats, neighb_inds, weights, W1, b1, W2,
           kernel_points):
    nb_flat = neighb_inds.astype(jnp.int32).reshape(-1)
    qp_flat = jnp.pad(q_pts, ((0, 0), (0, 1))).reshape(-1)
    spts_flat = s_pts.T.reshape(-1)
    wts_flat = _interleave_pairs(weights).reshape(-1)
    kp_flat = jnp.pad(kernel_points, ((0, 16 - K), (0, 0))).T.reshape(-1)
    W2p = jnp.pad(W2, ((0, 0), (0, MODW - K * CPG)))
    sf_pairs = jnp.pad(_interleave_pairs(s_feats), ((0, 0), (0, 64)))
    mod = _modulations(s_feats, W1, b1, W2p)
    return _sc_main(nb_flat, qp_flat, spts_flat, sf_pairs, wts_flat,
                    kp_flat, mod.reshape(-1))
